# baseline (speedup 1.0000x reference)
import functools

import jax
import jax.numpy as jnp
from jax import lax
from jax.experimental import pallas as pl
from jax.experimental.pallas import tpu as pltpu
from jax.experimental.pallas import tpu_sc as plsc

N = 1_000_000
NUM_CLASSES = 7
LANES = 16
NUM_CORES = 1
NUM_WORKERS = 16 * NUM_CORES
PER_W = 62_464
BUF_W = 4_096
CHUNKS = (4096,) * 15 + (1024,)
NCHUNKS = len(CHUNKS)
EXTRA_COL = NUM_WORKERS * PER_W
EXTRA_W = N - EXTRA_COL
EXTRA_ALIGNED = 512
TAIL_W = EXTRA_W - EXTRA_ALIGNED


UNROLL = 8


def _zero_scatter_pass(nums_ref, out_ref, map_ref, val_ref, ngroups):
    zeros16 = jnp.zeros((LANES,), jnp.float32)
    col0 = lax.iota(jnp.int32, LANES)

    def body(it, col):
        cls_l, val_l = [], []
        for u in range(UNROLL):
            g = it * UNROLL + u
            nums = nums_ref[pl.ds(g * LANES, LANES)]
            cls_l.append(plsc.load_gather(map_ref, [nums]))
            val_l.append(plsc.load_gather(val_ref, [nums]))
        for u in range(UNROLL):
            g = it * UNROLL + u
            for j in range(NUM_CLASSES):
                out_ref[j, pl.ds(g * LANES, LANES)] = zeros16
        for u in range(UNROLL):
            plsc.store_scatter(out_ref, [cls_l[u], col + u * LANES], val_l[u])
        return col + UNROLL * LANES

    lax.fori_loop(0, ngroups // UNROLL, body, col0)


def _scatter_save_pass(nums_ref, out_ref, cls_buf, map_ref, val_ref, ngroups):
    col0 = lax.iota(jnp.int32, LANES)

    def body(it, col):
        cls_l, val_l = [], []
        for u in range(UNROLL):
            g = it * UNROLL + u
            nums = nums_ref[pl.ds(g * LANES, LANES)]
            cls_l.append(plsc.load_gather(map_ref, [nums]))
            val_l.append(plsc.load_gather(val_ref, [nums]))
        for u in range(UNROLL):
            g = it * UNROLL + u
            cls_buf[pl.ds(g * LANES, LANES)] = cls_l[u]
        for u in range(UNROLL):
            plsc.store_scatter(out_ref, [cls_l[u], col + u * LANES], val_l[u])
        return col + UNROLL * LANES

    lax.fori_loop(0, ngroups // UNROLL, body, col0)


def _clear_pass(out_ref, cls_buf, ngroups):
    zeros16 = jnp.zeros((LANES,), jnp.float32)
    col0 = lax.iota(jnp.int32, LANES)

    def body(it, col):
        cls_l = []
        for u in range(UNROLL):
            g = it * UNROLL + u
            cls_l.append(cls_buf[pl.ds(g * LANES, LANES)])
        for u in range(UNROLL):
            plsc.store_scatter(out_ref, [cls_l[u], col + u * LANES], zeros16)
        return col + UNROLL * LANES

    lax.fori_loop(0, ngroups // UNROLL, body, col0)


def _body(numbers_hbm, map_hbm, val_hbm, out_hbm,
          map_v, val_v, n0, n1, out0, out1, cls0, cls1, tail_n, tail_out,
          si0, si1, so0, so1, st0, st1):
    wid = lax.axis_index("s") * NUM_CORES + lax.axis_index("c")
    base = wid * PER_W

    pltpu.sync_copy(map_hbm, map_v)
    pltpu.sync_copy(val_hbm, val_v)

    nums_bufs = (n0, n1)
    out_bufs = (out0, out1)
    in_sems = (si0, si1)
    out_sems = (so0, so1)
    offs = []
    o = 0
    for w in CHUNKS:
        offs.append(o)
        o += w

    def issue_in(c):
        return pltpu.async_copy(
            numbers_hbm.at[pl.ds(base + offs[c], CHUNKS[c])],
            nums_bufs[c % 2].at[pl.ds(0, CHUNKS[c])], in_sems[c % 2])

    def issue_out(c):
        w = CHUNKS[c]
        return pltpu.async_copy(
            out_bufs[c % 2].at[:, pl.ds(0, w)],
            out_hbm.at[:, pl.ds(base + offs[c], w)], out_sems[c % 2])

    in_cps = [issue_in(0), issue_in(1)]
    out_cps = [None, None]
    cls_bufs = (cls0, cls1)
    is_last = wid == NUM_WORKERS - 1

    @pl.when(is_last)
    def _tail_prefetch():
        pltpu.async_copy(numbers_hbm.at[pl.ds(EXTRA_COL, EXTRA_W)],
                         tail_n, st0)

    zeros16 = jnp.zeros((LANES,), jnp.float32)

    def _zero_buf(buf):
        def zbody(z, _):
            for j in range(NUM_CLASSES):
                buf[j, pl.ds(z * LANES, LANES)] = zeros16
            return 0
        lax.fori_loop(0, BUF_W // LANES, zbody, 0)

    _zero_buf(out0)
    for c in range(NCHUNKS):
        b = c % 2
        in_cps[b].wait()
        if out_cps[b] is not None:
            out_cps[b].wait()
            _clear_pass(out_bufs[b], cls_bufs[b], CHUNKS[c - 2] // LANES)
        _scatter_save_pass(nums_bufs[b], out_bufs[b], cls_bufs[b],
                           map_v, val_v, CHUNKS[c] // LANES)
        out_cps[b] = issue_out(c)
        if c == 0:
            _zero_buf(out1)
        if c + 2 < NCHUNKS:
            in_cps[b] = issue_in(c + 2)

    @pl.when(is_last)
    def _tail_edge():
        pltpu.make_async_copy(numbers_hbm.at[pl.ds(EXTRA_COL, EXTRA_W)],
                              tail_n, st0).wait()

        def tbody(g, col):
            nums = tail_n[pl.ds(EXTRA_ALIGNED + g * LANES, LANES)]
            cls = plsc.load_gather(map_v, [nums])
            val = plsc.load_gather(val_v, [nums])
            for j in range(NUM_CLASSES):
                tail_out[j, pl.ds(g * LANES, LANES)] = zeros16
            plsc.store_scatter(tail_out, [cls, col], val)
            return col + LANES

        lax.fori_loop(0, TAIL_W // LANES, tbody, lax.iota(jnp.int32, LANES))
        pltpu.async_copy(
            tail_out,
            out_hbm.at[:, pl.ds(EXTRA_COL + EXTRA_ALIGNED, TAIL_W)], st1)

    out_cps[0].wait()

    @pl.when(is_last)
    def _tail_aligned():
        _zero_scatter_pass(tail_n, out0, map_v, val_v,
                           EXTRA_ALIGNED // LANES)
        pltpu.async_copy(out0.at[:, pl.ds(0, EXTRA_ALIGNED)],
                         out_hbm.at[:, pl.ds(EXTRA_COL, EXTRA_ALIGNED)],
                         st0).wait()
        pltpu.make_async_copy(
            tail_out,
            out_hbm.at[:, pl.ds(EXTRA_COL + EXTRA_ALIGNED, TAIL_W)],
            st1).wait()

    out_cps[1].wait()


@jax.jit
def _onehot_sc(numbers, map_tab, val_tab):
    mesh = plsc.VectorSubcoreMesh(core_axis_name="c", subcore_axis_name="s", num_cores=NUM_CORES)
    run = functools.partial(
        pl.kernel,
        out_type=jax.ShapeDtypeStruct((NUM_CLASSES, N), jnp.float32),
        mesh=mesh,
        scratch_types=[
            pltpu.VMEM((32,), jnp.int32),
            pltpu.VMEM((32,), jnp.float32),
            pltpu.VMEM((BUF_W,), jnp.int32),
            pltpu.VMEM((BUF_W,), jnp.int32),
            pltpu.VMEM((NUM_CLASSES, BUF_W), jnp.float32),
            pltpu.VMEM((NUM_CLASSES, BUF_W), jnp.float32),
            pltpu.VMEM((BUF_W,), jnp.int32),
            pltpu.VMEM((BUF_W,), jnp.int32),
            pltpu.VMEM((EXTRA_W,), jnp.int32),
            pltpu.VMEM((NUM_CLASSES, TAIL_W), jnp.float32),
            pltpu.SemaphoreType.DMA,
            pltpu.SemaphoreType.DMA,
            pltpu.SemaphoreType.DMA,
            pltpu.SemaphoreType.DMA,
            pltpu.SemaphoreType.DMA,
            pltpu.SemaphoreType.DMA,
        ],
        compiler_params=pltpu.CompilerParams(needs_layout_passes=False),
    )(_body)
    return run(numbers, map_tab, val_tab)


def kernel(numbers, mapping, eye):
    map_tab = jnp.zeros((32,), jnp.int32).at[: mapping.shape[0]].set(mapping)
    val_tab = jnp.diagonal(eye)[map_tab]
    out_t = _onehot_sc(numbers, map_tab, val_tab)
    return out_t.T


# speedup vs baseline: 1.3157x; 1.3157x over previous
import functools

import jax
import jax.numpy as jnp
from jax import lax
from jax.experimental import pallas as pl
from jax.experimental.pallas import tpu as pltpu
from jax.experimental.pallas import tpu_sc as plsc

N = 1_000_000
NUM_CLASSES = 7
LANES = 16
NUM_WORKERS = 32
PER_W = 31_232
BUF_W = 6_144
CHUNKS = (6144, 6144, 6144, 6144, 6144, 512)
NCHUNKS = len(CHUNKS)
EXTRA_COL = NUM_WORKERS * PER_W
EXTRA_W = N - EXTRA_COL
EXTRA_ALIGNED = 512
TAIL_W = EXTRA_W - EXTRA_ALIGNED


UNROLL = 8


def _zero_scatter_pass(nums_ref, out_ref, map_ref, val_ref, ngroups):
    zeros16 = jnp.zeros((LANES,), jnp.float32)
    col0 = lax.iota(jnp.int32, LANES)

    def body(it, col):
        cls_l, val_l = [], []
        for u in range(UNROLL):
            g = it * UNROLL + u
            nums = nums_ref[pl.ds(g * LANES, LANES)]
            cls_l.append(plsc.load_gather(map_ref, [nums]))
            val_l.append(plsc.load_gather(val_ref, [nums]))
        for u in range(UNROLL):
            g = it * UNROLL + u
            for j in range(NUM_CLASSES):
                out_ref[j, pl.ds(g * LANES, LANES)] = zeros16
        for u in range(UNROLL):
            plsc.store_scatter(out_ref, [cls_l[u], col + u * LANES], val_l[u])
        return col + UNROLL * LANES

    lax.fori_loop(0, ngroups // UNROLL, body, col0)


def _scatter_save_pass(nums_ref, out_ref, cls_buf, map_ref, val_ref, ngroups):
    col0 = lax.iota(jnp.int32, LANES)

    def body(it, col):
        cls_l, val_l = [], []
        for u in range(UNROLL):
            g = it * UNROLL + u
            nums = nums_ref[pl.ds(g * LANES, LANES)]
            cls_l.append(plsc.load_gather(map_ref, [nums]))
            val_l.append(plsc.load_gather(val_ref, [nums]))
        for u in range(UNROLL):
            g = it * UNROLL + u
            cls_buf[pl.ds(g * LANES, LANES)] = cls_l[u]
        for u in range(UNROLL):
            plsc.store_scatter(out_ref, [cls_l[u], col + u * LANES], val_l[u])
        return col + UNROLL * LANES

    lax.fori_loop(0, ngroups // UNROLL, body, col0)


def _clear_pass(out_ref, cls_buf, ngroups):
    zeros16 = jnp.zeros((LANES,), jnp.float32)
    col0 = lax.iota(jnp.int32, LANES)

    def body(it, col):
        cls_l = []
        for u in range(UNROLL):
            g = it * UNROLL + u
            cls_l.append(cls_buf[pl.ds(g * LANES, LANES)])
        for u in range(UNROLL):
            plsc.store_scatter(out_ref, [cls_l[u], col + u * LANES], zeros16)
        return col + UNROLL * LANES

    lax.fori_loop(0, ngroups // UNROLL, body, col0)


def _body(numbers_hbm, map_hbm, val_hbm, out_hbm,
          map_v, val_v, n0, n1, out0, out1, cls0, cls1, tail_n, tail_out,
          si0, si1, so0, so1, st0, st1):
    wid = lax.axis_index("s") * 2 + lax.axis_index("c")
    base = wid * PER_W

    pltpu.sync_copy(map_hbm, map_v)
    pltpu.sync_copy(val_hbm, val_v)

    nums_bufs = (n0, n1)
    out_bufs = (out0, out1)
    in_sems = (si0, si1)
    out_sems = (so0, so1)
    offs = []
    o = 0
    for w in CHUNKS:
        offs.append(o)
        o += w

    def issue_in(c):
        return pltpu.async_copy(
            numbers_hbm.at[pl.ds(base + offs[c], CHUNKS[c])],
            nums_bufs[c % 2].at[pl.ds(0, CHUNKS[c])], in_sems[c % 2])

    def issue_out(c):
        w = CHUNKS[c]
        return pltpu.async_copy(
            out_bufs[c % 2].at[:, pl.ds(0, w)],
            out_hbm.at[:, pl.ds(base + offs[c], w)], out_sems[c % 2])

    in_cps = [issue_in(0), issue_in(1)]
    out_cps = [None, None]
    cls_bufs = (cls0, cls1)
    is_last = wid == NUM_WORKERS - 1

    @pl.when(is_last)
    def _tail_prefetch():
        pltpu.async_copy(numbers_hbm.at[pl.ds(EXTRA_COL, EXTRA_W)],
                         tail_n, st0)

    zeros16 = jnp.zeros((LANES,), jnp.float32)

    def _zero_buf(buf):
        def zbody(z, _):
            for j in range(NUM_CLASSES):
                buf[j, pl.ds(z * LANES, LANES)] = zeros16
            return 0
        lax.fori_loop(0, BUF_W // LANES, zbody, 0)

    _zero_buf(out0)
    for c in range(NCHUNKS):
        b = c % 2
        in_cps[b].wait()
        if out_cps[b] is not None:
            out_cps[b].wait()
            _clear_pass(out_bufs[b], cls_bufs[b], CHUNKS[c - 2] // LANES)
        _scatter_save_pass(nums_bufs[b], out_bufs[b], cls_bufs[b],
                           map_v, val_v, CHUNKS[c] // LANES)
        out_cps[b] = issue_out(c)
        if c == 0:
            _zero_buf(out1)
        if c + 2 < NCHUNKS:
            in_cps[b] = issue_in(c + 2)

    @pl.when(is_last)
    def _tail_edge():
        pltpu.make_async_copy(numbers_hbm.at[pl.ds(EXTRA_COL, EXTRA_W)],
                              tail_n, st0).wait()

        def tbody(g, col):
            nums = tail_n[pl.ds(EXTRA_ALIGNED + g * LANES, LANES)]
            cls = plsc.load_gather(map_v, [nums])
            val = plsc.load_gather(val_v, [nums])
            for j in range(NUM_CLASSES):
                tail_out[j, pl.ds(g * LANES, LANES)] = zeros16
            plsc.store_scatter(tail_out, [cls, col], val)
            return col + LANES

        lax.fori_loop(0, TAIL_W // LANES, tbody, lax.iota(jnp.int32, LANES))
        pltpu.async_copy(
            tail_out,
            out_hbm.at[:, pl.ds(EXTRA_COL + EXTRA_ALIGNED, TAIL_W)], st1)

    out_cps[0].wait()

    @pl.when(is_last)
    def _tail_aligned():
        _zero_scatter_pass(tail_n, out0, map_v, val_v,
                           EXTRA_ALIGNED // LANES)
        pltpu.async_copy(out0.at[:, pl.ds(0, EXTRA_ALIGNED)],
                         out_hbm.at[:, pl.ds(EXTRA_COL, EXTRA_ALIGNED)],
                         st0).wait()
        pltpu.make_async_copy(
            tail_out,
            out_hbm.at[:, pl.ds(EXTRA_COL + EXTRA_ALIGNED, TAIL_W)],
            st1).wait()

    out_cps[1].wait()


@jax.jit
def _onehot_sc(numbers, map_tab, val_tab):
    mesh = plsc.VectorSubcoreMesh(core_axis_name="c", subcore_axis_name="s")
    run = functools.partial(
        pl.kernel,
        out_type=jax.ShapeDtypeStruct((NUM_CLASSES, N), jnp.float32),
        mesh=mesh,
        scratch_types=[
            pltpu.VMEM((32,), jnp.int32),
            pltpu.VMEM((32,), jnp.float32),
            pltpu.VMEM((BUF_W,), jnp.int32),
            pltpu.VMEM((BUF_W,), jnp.int32),
            pltpu.VMEM((NUM_CLASSES, BUF_W), jnp.float32),
            pltpu.VMEM((NUM_CLASSES, BUF_W), jnp.float32),
            pltpu.VMEM((BUF_W,), jnp.int32),
            pltpu.VMEM((BUF_W,), jnp.int32),
            pltpu.VMEM((EXTRA_W,), jnp.int32),
            pltpu.VMEM((NUM_CLASSES, TAIL_W), jnp.float32),
            pltpu.SemaphoreType.DMA,
            pltpu.SemaphoreType.DMA,
            pltpu.SemaphoreType.DMA,
            pltpu.SemaphoreType.DMA,
            pltpu.SemaphoreType.DMA,
            pltpu.SemaphoreType.DMA,
        ],
        compiler_params=pltpu.CompilerParams(needs_layout_passes=False),
    )(_body)
    return run(numbers, map_tab, val_tab)


def kernel(numbers, mapping, eye):
    map_tab = jnp.zeros((32,), jnp.int32).at[: mapping.shape[0]].set(mapping)
    val_tab = jnp.diagonal(eye)[map_tab]
    out_t = _onehot_sc(numbers, map_tab, val_tab)
    return out_t.T
